# baseline (speedup 1.0000x reference)
import functools

import jax
import jax.numpy as jnp
from jax import lax
from jax.experimental import pallas as pl
from jax.experimental.pallas import tpu as pltpu
from jax.experimental.pallas import tpu_sc as plsc

F = 26
V = 100000
D = 32
P = 128
B = 16384

NC = 2
NS = 16
NW = NC * NS
BPW = B // NW
CB = 8
NCH = BPW // CB
CR = CB * F
NPK = CR // 16
HSTR = CR // 2
GROUPS = F * V // 4
ORPB = (F * D + P - 1) // P + 1


def _sc_gather_pack(tab_hbm, idx_hbm, off_hbm, dr_hbm, dc_hbm):
    mesh = plsc.VectorSubcoreMesh(core_axis_name="c", subcore_axis_name="s")

    @functools.partial(
        pl.kernel,
        mesh=mesh,
        out_type=jax.ShapeDtypeStruct((B * ORPB, 128), jnp.float32),
        compiler_params=pltpu.CompilerParams(
            use_tc_tiling_on_sc=False, needs_layout_passes=False),
        scratch_types=[
            pltpu.VMEM((NCH, 256), jnp.int32),
            pltpu.VMEM((NCH, 256), jnp.int32),
            pltpu.VMEM((256,), jnp.int32),
            pltpu.VMEM((256,), jnp.int32),
            pltpu.VMEM((CR, 128), jnp.float32),
            pltpu.VMEM((CR, 128), jnp.float32),
            pltpu.VMEM((CB * ORPB, 128), jnp.float32),
            pltpu.VMEM((CB * ORPB, 128), jnp.float32),
            pltpu.SemaphoreType.DMA,
            pltpu.SemaphoreType.DMA,
            pltpu.SemaphoreType.DMA,
            pltpu.SemaphoreType.DMA,
        ],
    )
    def k(tab, idx, off, dr, dc, out, idx_v, off_v, dr_v, dc_v,
          g0, g1, p0, p1, gs0, gs1, os0, os1):
        wid = lax.axis_index("s") * NC + lax.axis_index("c")
        pltpu.sync_copy(idx.at[wid], idx_v)
        pltpu.sync_copy(off.at[wid], off_v)
        pltpu.sync_copy(dr, dr_v)
        pltpu.sync_copy(dc, dc_v)

        gbufs = (g0, g1)
        pbufs = (p0, p1)
        gsems = (gs0, gs1)
        osems = (os0, os1)
        out_base = wid * BPW * ORPB

        def gather_cp(c, slot, s):
            return pltpu.async_copy(
                tab.at[idx_v.at[c, pl.ds(s * HSTR, HSTR)]],
                gbufs[slot].at[pl.ds(s * HSTR, HSTR)],
                gsems[slot],
            )

        def out_cp(c, slot):
            return pltpu.make_async_copy(
                pbufs[slot],
                out.at[pl.ds(out_base + c * CB * ORPB, CB * ORPB)],
                osems[slot],
            )

        def fire_gather(c, slot):
            for s in range(2):
                gather_cp(c, slot, s)

        def wait_gather(c, slot):
            for s in range(2):
                pltpu.make_async_copy(
                    tab.at[idx_v.at[c, pl.ds(s * HSTR, HSTR)]],
                    gbufs[slot].at[pl.ds(s * HSTR, HSTR)],
                    gsems[slot],
                ).wait()

        def select(c, slot):
            gb = gbufs[slot]
            pb = pbufs[slot]
            lane = lax.iota(jnp.int32, 16)
            for kpk in range(NPK):
                rows = lane + (kpk * 16)
                o16 = off_v[c, pl.ds(kpk * 16, 16)]
                dr16 = dr_v[pl.ds(kpk * 16, 16)]
                dc16 = dc_v[pl.ds(kpk * 16, 16)]
                col0 = o16 * D
                for j in range(D):
                    v = plsc.load_gather(gb, [rows, col0 + j])
                    plsc.store_scatter(pb, [dr16, dc16 + j], v)

        fire_gather(0, 0)

        def body(g, _):
            for b in range(2):
                c = g * 2 + b

                @pl.when(c + 1 < NCH)
                def _():
                    fire_gather(c + 1, (b + 1) % 2)

                wait_gather(c, b)

                @pl.when(c >= 2)
                def _():
                    out_cp(c - 2, b).wait()

                select(c, b)
                pltpu.async_copy(
                    pbufs[b],
                    out.at[pl.ds(out_base + c * CB * ORPB, CB * ORPB)],
                    osems[b],
                )
            return 0

        lax.fori_loop(0, NCH // 2, body, 0, unroll=False)
        out_cp(NCH - 2, 0).wait()
        out_cp(NCH - 1, 1).wait()

    return k(tab_hbm, idx_hbm, off_hbm, dr_hbm, dc_hbm)


def _tc_body(x_ref, m_ref, g_ref, bt_ref, w1_ref, b1_ref, w2_ref, b2_ref,
             o_ref):
    bt = o_ref.shape[0]
    x = x_ref[...]
    row_j = lax.broadcasted_iota(jnp.int32, x.shape, 0) % ORPB
    lane = lax.broadcasted_iota(jnp.int32, x.shape, 1)
    valid = row_j * 128 + lane < F * D
    x = jnp.where(valid, x, 0.0)
    m_m = m_ref[...]
    mu = jnp.dot(x.astype(jnp.bfloat16), m_m,
                 preferred_element_type=jnp.float32)
    m2 = jnp.dot((x * x).astype(jnp.bfloat16), m_m,
                 preferred_element_type=jnp.float32)
    var = m2 - mu * mu
    h = (x - mu) * lax.rsqrt(var + 1e-5)
    gt = jnp.broadcast_to(g_ref[...][None], (bt, ORPB, 128)).reshape(x.shape)
    btl = jnp.broadcast_to(bt_ref[...][None], (bt, ORPB, 128)).reshape(x.shape)
    h = h * gt + btl
    h3 = h.astype(jnp.bfloat16).reshape(bt, ORPB, 128)
    w1 = w1_ref[...]
    t = b1_ref[...].astype(jnp.float32)
    for j in range(ORPB):
        t = t + jnp.dot(h3[:, j, :], w1[j], preferred_element_type=jnp.float32)
    u = 0.5 * t * (1.0 + lax.erf(t * 0.7071067811865476))
    o_ref[...] = jnp.dot(u.astype(jnp.bfloat16), w2_ref[...],
                         preferred_element_type=jnp.float32) + b2_ref[...]


def _tc_mlp(emb4, m_m, gamma, beta, w1p, b1, w2, b2, interpret=False):
    BT = 256
    grid = (B // BT,)
    return pl.pallas_call(
        _tc_body,
        grid=grid,
        in_specs=[
            pl.BlockSpec((BT * ORPB, 128), lambda i: (i, 0)),
            pl.BlockSpec((128, 128), lambda i: (0, 0)),
            pl.BlockSpec((ORPB, 128), lambda i: (0, 0)),
            pl.BlockSpec((ORPB, 128), lambda i: (0, 0)),
            pl.BlockSpec((ORPB, 128, P), lambda i: (0, 0, 0)),
            pl.BlockSpec((1, P), lambda i: (0, 0)),
            pl.BlockSpec((P, P), lambda i: (0, 0)),
            pl.BlockSpec((1, P), lambda i: (0, 0)),
        ],
        out_specs=pl.BlockSpec((BT, P), lambda i: (i, 0)),
        out_shape=jax.ShapeDtypeStruct((B, P), jnp.float32),
        interpret=interpret,
    )(emb4, m_m, gamma, beta, w1p, b1, w2, b2)


def _mean_mat():
    lanes = jnp.arange(128, dtype=jnp.int32)
    m = jnp.where(lanes[:, None] // D == lanes[None, :] // D, 1.0 / D, 0.0)
    return m.astype(jnp.bfloat16)


def _pad_fd(a):
    return jnp.concatenate(
        [a.reshape(-1), jnp.zeros((ORPB * 128 - F * D,), a.dtype)]
    ).reshape(ORPB, 128)


def kernel(x_cat, tables, ln_gamma, ln_beta, W1, b1, W2, b2):
    flat = x_cat + (jnp.arange(F, dtype=jnp.int32) * V)[None, :]
    gidx = (flat // 4).reshape(NW, NCH, CR)
    off = (flat % 4).reshape(NW, NCH, CR)
    zpad = jnp.zeros((NW, NCH, 256 - CR), jnp.int32)
    gidx = jnp.concatenate([gidx, zpad], axis=-1)
    off = jnp.concatenate([off, zpad], axis=-1)
    i = jnp.arange(256, dtype=jnp.int32)
    dr = jnp.where(i < CR, (i // F) * ORPB + (i % F) // 4, 0)
    dc = jnp.where(i < CR, ((i % F) % 4) * D, 0)
    tab4 = tables.reshape(GROUPS, 128)

    emb4 = _sc_gather_pack(tab4, gidx, off, dr, dc)

    w1p = jnp.concatenate(
        [W1, jnp.zeros((ORPB * 128 - F * D, P), W1.dtype)], axis=0
    ).reshape(ORPB, 128, P).astype(jnp.bfloat16)
    return _tc_mlp(
        emb4, _mean_mat(), _pad_fd(ln_gamma), _pad_fd(ln_beta),
        w1p, b1.reshape(1, P), W2.astype(jnp.bfloat16), b2.reshape(1, P),
    )


# speedup vs baseline: 1.2871x; 1.2871x over previous
import functools

import jax
import jax.numpy as jnp
from jax import lax
from jax.experimental import pallas as pl
from jax.experimental.pallas import tpu as pltpu
from jax.experimental.pallas import tpu_sc as plsc

F = 26
V = 100000
D = 32
P = 128
B = 16384

NC = 2
NS = 16
NW = NC * NS
BPW = B // NW
NST = BPW // 128


def _sc_gather(tab_hbm, idx_hbm):
    mesh = plsc.VectorSubcoreMesh(core_axis_name="c", subcore_axis_name="s")

    @functools.partial(
        pl.kernel,
        mesh=mesh,
        out_type=jax.ShapeDtypeStruct((F, B, 128), jnp.float32),
        compiler_params=pltpu.CompilerParams(
            use_tc_tiling_on_sc=False, needs_layout_passes=False),
        scratch_types=[
            pltpu.VMEM((F, BPW), jnp.int32),
        ] + [pltpu.VMEM((128, D), jnp.float32) for _ in range(8)]
          + [pltpu.SemaphoreType.DMA for _ in range(16)],
    )
    def k(tab, idx, out, idx_v, *bufs_sems):
        bufs = bufs_sems[:8]
        gsems = bufs_sems[8:16]
        osems = bufs_sems[16:24]
        wid = lax.axis_index("s") * NC + lax.axis_index("c")
        pltpu.sync_copy(idx.at[wid], idx_v)
        b0 = wid * BPW

        def gather_cp(f, slot, s):
            return pltpu.make_async_copy(
                tab.at[f].at[idx_v.at[f, pl.ds(s * 128, 128)]],
                bufs[slot * 4 + s],
                gsems[slot * 4 + s],
            )

        def out_cp(f, slot, s):
            return pltpu.make_async_copy(
                bufs[slot * 4 + s],
                out.at[f, pl.ds(b0 + s * 128, 128), pl.ds(0, D)],
                osems[slot * 4 + s],
            )

        def fire_gather(f, slot):
            for s in range(NST):
                pltpu.async_copy(
                    tab.at[f].at[idx_v.at[f, pl.ds(s * 128, 128)]],
                    bufs[slot * 4 + s],
                    gsems[slot * 4 + s],
                )

        fire_gather(0, 0)
        fire_gather(1, 1)

        def body(g, _):
            for p in range(2):
                f = g * 2 + p
                slot = p
                nslot = (p + 1) % 2

                @pl.when(jnp.logical_and(f >= 1, f + 1 < F))
                def _():
                    for s in range(NST):
                        out_cp(f - 1, nslot, s).wait()
                    fire_gather(f + 1, nslot)

                for s in range(NST):
                    gather_cp(f, slot, s).wait()
                    pltpu.async_copy(
                        bufs[slot * 4 + s],
                        out.at[f, pl.ds(b0 + s * 128, 128), pl.ds(0, D)],
                        osems[slot * 4 + s],
                    )
            return 0

        lax.fori_loop(0, F // 2, body, 0)
        for s in range(NST):
            out_cp(F - 2, 0, s).wait()
            out_cp(F - 1, 1, s).wait()

    return k(tab_hbm, idx_hbm)


def _tc_body(x_ref, g_ref, bt_ref, w1_ref, b1_ref, w2_ref, b2_ref, o_ref):
    bt = o_ref.shape[0]
    x = x_ref[...]
    lane = lax.broadcasted_iota(jnp.int32, x.shape, 2)
    x = jnp.where(lane < D, x, 0.0)
    mu = jnp.sum(x, axis=2, keepdims=True) * (1.0 / D)
    m2 = jnp.sum(x * x, axis=2, keepdims=True) * (1.0 / D)
    var = m2 - mu * mu
    h = (x - mu) * lax.rsqrt(var + 1e-5)
    h = h * g_ref[...][:, None, :] + bt_ref[...][:, None, :]
    hb = h.astype(jnp.bfloat16)
    w1 = w1_ref[...]
    t = b1_ref[...].astype(jnp.float32)
    for f in range(F):
        t = t + jnp.dot(hb[f], w1[f], preferred_element_type=jnp.float32)
    u = 0.5 * t * (1.0 + lax.erf(t * 0.7071067811865476))
    o_ref[...] = jnp.dot(u.astype(jnp.bfloat16), w2_ref[...],
                         preferred_element_type=jnp.float32) + b2_ref[...]


def _tc_mlp(emb, gamma, beta, w1p, b1, w2, b2, interpret=False):
    BT = 128
    grid = (B // BT,)
    return pl.pallas_call(
        _tc_body,
        grid=grid,
        in_specs=[
            pl.BlockSpec((F, BT, 128), lambda i: (0, i, 0)),
            pl.BlockSpec((F, 128), lambda i: (0, 0)),
            pl.BlockSpec((F, 128), lambda i: (0, 0)),
            pl.BlockSpec((F, 128, P), lambda i: (0, 0, 0)),
            pl.BlockSpec((1, P), lambda i: (0, 0)),
            pl.BlockSpec((P, P), lambda i: (0, 0)),
            pl.BlockSpec((1, P), lambda i: (0, 0)),
        ],
        out_specs=pl.BlockSpec((BT, P), lambda i: (i, 0)),
        out_shape=jax.ShapeDtypeStruct((B, P), jnp.float32),
        interpret=interpret,
    )(emb, gamma, beta, w1p, b1, w2, b2)


def _pad_lanes(a):
    return jnp.concatenate([a, jnp.zeros((F, 128 - D), a.dtype)], axis=1)


def kernel(x_cat, tables, ln_gamma, ln_beta, W1, b1, W2, b2):
    idx = x_cat.reshape(NW, BPW, F).transpose(0, 2, 1)
    emb = _sc_gather(tables, idx)
    w1p = jnp.pad(W1.reshape(F, D, P), ((0, 0), (0, 128 - D), (0, 0)))
    return _tc_mlp(
        emb, _pad_lanes(ln_gamma), _pad_lanes(ln_beta),
        w1p.astype(jnp.bfloat16), b1.reshape(1, P),
        W2.astype(jnp.bfloat16), b2.reshape(1, P),
    )
